# baseline (speedup 1.0000x reference)
import functools

import jax
import jax.numpy as jnp
from jax import lax
from jax.experimental import pallas as pl
from jax.experimental.pallas import tpu as pltpu
from jax.experimental.pallas import tpu_sc as plsc

N = 100000
E = 1600000
XD = 28
PD = 3
ED = 7
NG = 2048
NF = 25000
NKER = {1: 10, 2: 20, 3: 30, 4: 50}
GED = 32

C_G = 392
C_S = 112
SEL_P = 25088
NEI_P = {1: 25088, 2: 50176, 3: 75264, 4: 100352}
PRE_ROWS = 50176
DUMMY_NODE = 50100
ACC_ROWS = 2176
DUMMY_SEG = 2048

_MESH = plsc.VectorSubcoreMesh(core_axis_name="c", subcore_axis_name="s")
_SC_PARAMS = pltpu.CompilerParams(use_tc_tiling_on_sc=False)


def _stats_body(x_ref, s1_ref, s2_ref):
    @pl.when(pl.program_id(0) == 0)
    def _():
        s1_ref[...] = jnp.zeros_like(s1_ref)
        s2_ref[...] = jnp.zeros_like(s2_ref)

    v = x_ref[...]
    s1_ref[...] += jnp.sum(v, axis=0, keepdims=True)
    s2_ref[...] += jnp.sum(v * v, axis=0, keepdims=True)


def _flat_stats(arr2d, block_rows):
    rows, width = arr2d.shape
    grid = rows // block_rows
    return pl.pallas_call(
        _stats_body,
        grid=(grid,),
        in_specs=[pl.BlockSpec((block_rows, width), lambda i: (i, 0))],
        out_specs=[pl.BlockSpec((1, width), lambda i: (0, 0)),
                   pl.BlockSpec((1, width), lambda i: (0, 0))],
        out_shape=[jax.ShapeDtypeStruct((1, width), jnp.float32),
                   jax.ShapeDtypeStruct((1, width), jnp.float32)],
    )(arr2d)


def _gather_body(nt_hbm, et_hbm, dst_hbm,
                 n1, n2, n3, n4, s1, s2, s3, s4,
                 eag1, xng1, eag2, xng2, eag3, xng3, eag4, xng4,
                 xpsg1, xpsg2, xpsg3, xpsg4,
                 idxv, eav, dstv, xnv):
    wid = lax.axis_index("s") * 2 + lax.axis_index("c")

    nei = ((n1, eag1, xng1, 2), (n2, eag2, xng2, 4),
           (n3, eag3, xng3, 6), (n4, eag4, xng4, 8))
    for nidx_hbm, eag_hbm, xng_hbm, trips in nei:
        base = wid * (trips * C_G)

        @pl.loop(0, trips)
        def _(j):
            off = base + j * C_G
            pltpu.sync_copy(nidx_hbm.at[pl.ds(off, C_G)], idxv)
            pltpu.sync_copy(et_hbm.at[idxv], eav)
            pltpu.sync_copy(dst_hbm.at[idxv], dstv)
            pltpu.sync_copy(nt_hbm.at[dstv], xnv)
            pltpu.sync_copy(eav, eag_hbm.at[pl.ds(off, C_G)])
            pltpu.sync_copy(xnv, xng_hbm.at[pl.ds(off, C_G)])

    for sel_hbm, xpsg_hbm in ((s1, xpsg1), (s2, xpsg2), (s3, xpsg3),
                              (s4, xpsg4)):
        base = wid * (2 * C_G)

        @pl.loop(0, 2)
        def _(j):
            off = base + j * C_G
            pltpu.sync_copy(sel_hbm.at[pl.ds(off, C_G)], idxv)
            pltpu.sync_copy(nt_hbm.at[idxv], xnv)
            pltpu.sync_copy(xnv, xpsg_hbm.at[pl.ds(off, C_G)])


_sc_gather = functools.partial(
    pl.kernel,
    _gather_body,
    out_type=[t for k in (1, 2, 3, 4) for t in
              (jax.ShapeDtypeStruct((NEI_P[k], 8), jnp.float32),
               jax.ShapeDtypeStruct((NEI_P[k], 32), jnp.float32))] +
             [jax.ShapeDtypeStruct((SEL_P, 32), jnp.float32)
              for _ in (1, 2, 3, 4)],
    mesh=_MESH,
    scratch_types=[pltpu.VMEM((C_G,), jnp.int32),
                   pltpu.VMEM((C_G, 8), jnp.float32),
                   pltpu.VMEM((C_G,), jnp.int32),
                   pltpu.VMEM((C_G, 32), jnp.float32)],
    compiler_params=_SC_PARAMS,
)()


def _dense_body(k, nk, bn,
                ax_ref, ae_ref, wx_ref, wp_ref, we_ref, wfx_ref, wfp_ref,
                bke_ref, w1k_ref,
                neix_ref, neip_ref, neie_ref, xng_ref, eag_ref,
                xf_ref, pf_ref, xpsg_ref, z_ref):
    ax = ax_ref[...]
    ae = ae_ref[...]
    f32 = jnp.float32
    xng = xng_ref[...][:, :XD]
    eag = eag_ref[...][:, :ED]
    xpsg = xpsg_ref[...]
    nb = jnp.dot(neix_ref[...] + xng * ax, wx_ref[...],
                 preferred_element_type=f32)
    nb += jnp.dot(neip_ref[...], wp_ref[...], preferred_element_type=f32)
    nb += jnp.dot(neie_ref[...] + eag * ae, we_ref[...],
                  preferred_element_type=f32)
    if k > 1:
        nb = nb.reshape(bn, k, nk).sum(axis=1)
    s = nb + jnp.dot(xf_ref[...] + xpsg[:, :XD] * ax, wfx_ref[...],
                     preferred_element_type=f32)
    s += jnp.dot(pf_ref[...] + xpsg[:, XD:XD + PD], wfp_ref[...],
                 preferred_element_type=f32)
    s += bke_ref[...]
    s = s * jax.nn.sigmoid(s)
    z_ref[...] = jnp.dot(s, w1k_ref[...], preferred_element_type=f32)


def _dense(k, nk, ax, ae, wx, wp, we, wfx, wfp, bke, w1k,
           neix, neip, neie, xng, eag, xf, pf, xpsg):
    bn = 1000
    grid = NF // bn
    full = lambda shape: pl.BlockSpec(shape, lambda i: tuple(0 for _ in shape))
    row = lambda r, d: pl.BlockSpec((r, d), lambda i: (i, 0))
    return pl.pallas_call(
        functools.partial(_dense_body, k, nk, bn),
        grid=(grid,),
        in_specs=[full((1, XD)), full((1, ED)), full((XD, nk)),
                  full((PD, nk)), full((ED, nk)), full((XD, nk)),
                  full((PD, nk)), full((1, nk)), full((nk, GED)),
                  row(bn * k, XD), row(bn * k, PD), row(bn * k, ED),
                  row(bn * k, 32), row(bn * k, 8),
                  row(bn, XD), row(bn, PD), row(bn, 32)],
        out_specs=[pl.BlockSpec((bn, GED), lambda i: (i, 0))],
        out_shape=[jax.ShapeDtypeStruct((SEL_P, GED), jnp.float32)],
    )(ax, ae, wx, wp, we, wfx, wfp, bke, w1k,
      neix, neip, neie, xng, eag, xf, pf, xpsg)[0]


def _scatter_body(z1, z2, z3, z4, q1, q2, q3, q4, batch2, b1_hbm, out_hbm,
                  zv, iv, iv2, bv, b1v, pre, acc):
    c = lax.axis_index("c")
    t = lax.axis_index("s")
    base_n = c * 50000

    @pl.loop(0, C_S)
    def _(r):
        zero = jnp.zeros((16,), jnp.float32)
        zv[r, pl.ds(0, 16)] = zero
        zv[r, pl.ds(16, 16)] = zero

    pltpu.sync_copy(zv.at[pl.ds(0, C_S)],
                    acc.at[pl.ds(t * (ACC_ROWS // 16), C_S)])
    pltpu.sync_copy(zv.at[pl.ds(0, ACC_ROWS // 16 - C_S)],
                    acc.at[pl.ds(t * (ACC_ROWS // 16) + C_S,
                                 ACC_ROWS // 16 - C_S)])

    pltpu.sync_copy(b1_hbm, b1v)

    @pl.loop(0, C_S)
    def _(r):
        zv[r, pl.ds(0, 16)] = b1v[pl.ds(0, 16)]
        zv[r, pl.ds(16, 16)] = b1v[pl.ds(16, 16)]

    @pl.loop(0, 3136 // C_S)
    def _(j):
        pltpu.sync_copy(zv, pre.at[pl.ds(t * 3136 + j * C_S, C_S)])

    plsc.subcore_barrier()

    for z_hbm, q_hbm in ((z1, q1), (z2, q2), (z3, q3), (z4, q4)):
        @pl.loop(0, 1568 // C_S)
        def _(j):
            off = t * 1568 + j * C_S
            pltpu.sync_copy(z_hbm.at[pl.ds(off, C_S)], zv)
            pltpu.sync_copy(q_hbm.at[pl.ds(off, C_S)], iv)

            @pl.loop(0, C_S // 16)
            def _(u):
                v = iv[pl.ds(u * 16, 16)]
                loc = v - base_n
                ok = (loc >= 0) & (loc < 50000)
                iv2[pl.ds(u * 16, 16)] = jnp.where(ok, loc, DUMMY_NODE)

            pltpu.sync_copy(zv, pre.at[iv2], add=True)

    plsc.subcore_barrier()

    @pl.loop(0, 3136 // C_S)
    def _(j):
        off = t * 3136 + j * C_S
        pltpu.sync_copy(pre.at[pl.ds(off, C_S)], zv)
        pltpu.sync_copy(batch2.at[c, pl.ds(off, C_S)], bv)

        @pl.loop(0, C_S)
        def _(r):
            for h in (0, 16):
                v = zv[r, pl.ds(h, 16)]
                sg = 1.0 / (1.0 + jnp.exp(-v))
                zv[r, pl.ds(h, 16)] = v * sg

        pltpu.sync_copy(zv, acc.at[bv], add=True)

    plsc.subcore_barrier()

    pltpu.sync_copy(acc.at[pl.ds(t * 128, 128)],
                    out_hbm.at[c, pl.ds(t * 128, 128)])


_sc_scatter = functools.partial(
    pl.kernel,
    _scatter_body,
    out_type=jax.ShapeDtypeStruct((2, NG, GED), jnp.float32),
    mesh=_MESH,
    scratch_types=[pltpu.VMEM((C_S, GED), jnp.float32),
                   pltpu.VMEM((C_S,), jnp.int32),
                   pltpu.VMEM((C_S,), jnp.int32),
                   pltpu.VMEM((C_S,), jnp.int32),
                   pltpu.VMEM((GED,), jnp.float32),
                   pltpu.VMEM_SHARED((PRE_ROWS, GED), jnp.float32),
                   pltpu.VMEM_SHARED((ACC_ROWS, GED), jnp.float32)],
    compiler_params=_SC_PARAMS,
)()


def _final_body(p_ref, w2_ref, b2_ref, cnt_ref, o_ref):
    pooled = p_ref[0] + p_ref[1]
    o_ref[...] = (jnp.dot(pooled, w2_ref[...], preferred_element_type=jnp.float32)
                  + cnt_ref[...] * b2_ref[...])


def _final(partials, w2, b2, cnt):
    return pl.pallas_call(
        _final_body,
        out_shape=jax.ShapeDtypeStruct((NG, GED), jnp.float32),
    )(partials, w2, b2.reshape(1, GED), cnt.reshape(NG, 1))


def kernel(x, p, edge_index, edge_attr, batch, x_focal_deg1, p_focal_deg1, nei_x_deg1, nei_p_deg1, nei_edge_attr_deg1, selected_index_deg1, nei_index_deg1, x_focal_deg2, p_focal_deg2, nei_x_deg2, nei_p_deg2, nei_edge_attr_deg2, selected_index_deg2, nei_index_deg2, x_focal_deg3, p_focal_deg3, nei_x_deg3, nei_p_deg3, nei_edge_attr_deg3, selected_index_deg3, nei_index_deg3, x_focal_deg4, p_focal_deg4, nei_x_deg4, nei_p_deg4, nei_edge_attr_deg4, selected_index_deg4, nei_index_deg4, gamma_x, beta_x, gamma_e, beta_e, Wx1, Wp1, We1, Wfx1, Wfp1, bk1, Wx2, Wp2, We2, Wfx2, Wfp2, bk2, Wx3, Wp3, We3, Wfx3, Wfp3, bk3, Wx4, Wp4, We4, Wfx4, Wfp4, bk4, W1, b1, W2, b2):
    xf = {1: x_focal_deg1, 2: x_focal_deg2, 3: x_focal_deg3, 4: x_focal_deg4}
    pf = {1: p_focal_deg1, 2: p_focal_deg2, 3: p_focal_deg3, 4: p_focal_deg4}
    nx = {1: nei_x_deg1, 2: nei_x_deg2, 3: nei_x_deg3, 4: nei_x_deg4}
    np_ = {1: nei_p_deg1, 2: nei_p_deg2, 3: nei_p_deg3, 4: nei_p_deg4}
    ne = {1: nei_edge_attr_deg1, 2: nei_edge_attr_deg2, 3: nei_edge_attr_deg3,
          4: nei_edge_attr_deg4}
    sel = {1: selected_index_deg1, 2: selected_index_deg2,
           3: selected_index_deg3, 4: selected_index_deg4}
    nidx = {1: nei_index_deg1, 2: nei_index_deg2, 3: nei_index_deg3,
            4: nei_index_deg4}
    wx = {1: Wx1, 2: Wx2, 3: Wx3, 4: Wx4}
    wp = {1: Wp1, 2: Wp2, 3: Wp3, 4: Wp4}
    we = {1: We1, 2: We2, 3: We3, 4: We4}
    wfx = {1: Wfx1, 2: Wfx2, 3: Wfx3, 4: Wfx4}
    wfp = {1: Wfp1, 2: Wfp2, 3: Wfp3, 4: Wfp4}
    bk = {1: bk1, 2: bk2, 3: bk3, 4: bk4}

    dst = edge_index[1]

    nidx_p = {k: jnp.pad(nidx[k], (0, NEI_P[k] - NF * k)) for k in NKER}
    sel_gp = {k: jnp.pad(sel[k], (0, SEL_P - NF)) for k in NKER}
    sel_sp = {k: jnp.pad(sel[k], (0, SEL_P - NF), constant_values=N)
              for k in NKER}

    node_tab = jnp.concatenate(
        [x, p, jnp.zeros((N, 1), jnp.float32)], axis=1)
    edge_tab = jnp.concatenate(
        [edge_attr, lax.bitcast_convert_type(dst, jnp.float32)[:, None]],
        axis=1)

    g = _sc_gather(node_tab, edge_tab, dst,
                   nidx_p[1], nidx_p[2], nidx_p[3], nidx_p[4],
                   sel_gp[1], sel_gp[2], sel_gp[3], sel_gp[4])
    eag = {k: g[2 * (k - 1)] for k in NKER}
    xng = {k: g[2 * (k - 1) + 1] for k in NKER}
    xpsg = {k: g[8 + (k - 1)] for k in NKER}

    sx1, sx2 = _flat_stats(x.reshape(2000, 1400), 400)
    se1, se2 = _flat_stats(edge_attr.reshape(25000, 448), 1000)
    sum_x = sx1.reshape(-1, XD).sum(axis=0)
    sumsq_x = sx2.reshape(-1, XD).sum(axis=0)
    sum_e = se1.reshape(-1, ED).sum(axis=0)
    sumsq_e = se2.reshape(-1, ED).sum(axis=0)
    mu_x = sum_x / N
    var_x = sumsq_x / N - mu_x * mu_x
    mu_e = sum_e / E
    var_e = sumsq_e / E - mu_e * mu_e
    a_x = gamma_x / jnp.sqrt(var_x + 1e-5)
    b_x = beta_x - mu_x * a_x
    a_e = gamma_e / jnp.sqrt(var_e + 1e-5)
    b_e = beta_e - mu_e * a_e
    ax2 = a_x.reshape(1, XD)
    ae2 = a_e.reshape(1, ED)

    z = {}
    off = 0
    for k, nk in NKER.items():
        bke = (bk[k] + k * (b_x @ wx[k]) + k * (b_e @ we[k])
               + b_x @ wfx[k]).reshape(1, nk)
        w1k = W1[off:off + nk]
        off += nk
        z[k] = _dense(k, nk, ax2, ae2, wx[k], wp[k], we[k], wfx[k], wfp[k],
                      bke, w1k,
                      nx[k].reshape(NF * k, XD), np_[k].reshape(NF * k, PD),
                      ne[k].reshape(NF * k, ED), xng[k], eag[k],
                      xf[k], pf[k], xpsg[k])

    batch2 = jnp.pad(batch.reshape(2, N // 2), ((0, 0), (0, PRE_ROWS - N // 2)),
                     constant_values=DUMMY_SEG)

    partials = _sc_scatter(z[1], z[2], z[3], z[4],
                           sel_sp[1], sel_sp[2], sel_sp[3], sel_sp[4],
                           batch2, b1)

    cnt = jnp.diff(jnp.searchsorted(batch, jnp.arange(NG + 1))).astype(jnp.float32)
    return _final(partials, W2, b2, cnt)
